# Initial kernel scaffold; baseline (speedup 1.0000x reference)
import functools

import jax
import jax.numpy as jnp
from jax import lax
from jax.experimental import pallas as pl
from jax.experimental.pallas import tpu as pltpu
from jax.experimental.pallas import tpu_sc as plsc

N = 10000
E = 160000
D = 128
HEADS = 8
DIM = 16
NMOD = 3

NC = 2
NS = 16
CHUNK = 200
E_PER_WORKER = E // (NC * NS)
NCHUNK = E_PER_WORKER // CHUNK
ROWS_PER_TILE = N // NS
RB = 125


def _mm_body(x_ref, w_ref, o_ref):
    o_ref[...] = jnp.dot(x_ref[...], w_ref[...],
                         preferred_element_type=jnp.float32)


def _matmul(x, w):
    return pl.pallas_call(
        _mm_body,
        grid=(10,),
        in_specs=[
            pl.BlockSpec((1000, D), lambda i: (i, 0)),
            pl.BlockSpec((D, D), lambda i: (0, 0)),
        ],
        out_specs=pl.BlockSpec((1000, D), lambda i: (i, 0)),
        out_shape=jax.ShapeDtypeStruct((N, D), jnp.float32),
    )(x, w)


def _sc_body(xw_hbm, e0, e1, e2, a0, a1, a2, nw_hbm, out_hbm,
             src_v, dst_v, attn_v, rows_v, nw_v, bounce_v, acc, sem):
    c = lax.axis_index("c")
    s = lax.axis_index("s")

    pltpu.sync_copy(nw_hbm, nw_v)

    zeros16 = jnp.zeros((16,), jnp.float32)

    def _zrow(i, _):
        for j in range(D // 16):
            bounce_v[i, pl.ds(16 * j, 16)] = zeros16
        return 0

    lax.fori_loop(0, RB, _zrow, 0)
    row_base = s * ROWS_PER_TILE
    for k in range(ROWS_PER_TILE // RB):
        pltpu.sync_copy(bounce_v, acc.at[pl.ds(row_base + RB * k, RB)])
    plsc.subcore_barrier()

    edges = [e0, e1, e2]
    attns = [a0, a1, a2]
    for m in range(NMOD):
        e_hbm = edges[m]
        a_hbm = attns[m]
        nws = [plsc.load_gather(nw_v,
                                [jnp.full((16,), j * 8 + m, jnp.int32)])
               for j in range(HEADS)]
        base_e = c * (E // NC) + s * E_PER_WORKER

        def _chunk(k, _, e_hbm=e_hbm, a_hbm=a_hbm, nws=nws, base_e=base_e):
            start = base_e + k * CHUNK
            pltpu.sync_copy(e_hbm.at[0, pl.ds(start, CHUNK)], src_v)
            pltpu.sync_copy(e_hbm.at[1, pl.ds(start, CHUNK)], dst_v)
            pltpu.sync_copy(a_hbm.at[pl.ds(start, CHUNK), :], attn_v)
            pltpu.async_copy(xw_hbm.at[src_v], rows_v, sem).wait()

            def _edge(i, _):
                i16 = jnp.full((16,), i, jnp.int32)
                for j in range(HEADS):
                    w = plsc.load_gather(
                        attn_v, [i16, jnp.full((16,), j, jnp.int32)]) * nws[j]
                    rows_v[i, pl.ds(16 * j, 16)] = (
                        rows_v[i, pl.ds(16 * j, 16)] * w)
                return 0

            lax.fori_loop(0, CHUNK, _edge, 0)
            pltpu.sync_copy(rows_v, acc.at[dst_v], add=True)
            return 0

        lax.fori_loop(0, NCHUNK, _chunk, 0)

    plsc.subcore_barrier()
    for k in range(ROWS_PER_TILE // RB):
        pltpu.sync_copy(acc.at[pl.ds(row_base + RB * k, RB)], bounce_v)
        pltpu.sync_copy(bounce_v,
                        out_hbm.at[c, pl.ds(row_base + RB * k, RB)])


_sc_scatter = functools.partial(
    pl.kernel,
    out_type=jax.ShapeDtypeStruct((NC, N, D), jnp.float32),
    mesh=plsc.VectorSubcoreMesh(core_axis_name="c", subcore_axis_name="s"),
    scratch_types=[
        pltpu.VMEM((CHUNK,), jnp.int32),
        pltpu.VMEM((CHUNK,), jnp.int32),
        pltpu.VMEM((CHUNK, HEADS), jnp.float32),
        pltpu.VMEM((CHUNK, D), jnp.float32),
        pltpu.VMEM((64,), jnp.float32),
        pltpu.VMEM((RB, D), jnp.float32),
        pltpu.VMEM_SHARED((N, D), jnp.float32),
        pltpu.SemaphoreType.DMA,
    ],
)(_sc_body)


def _comb_body(p_ref, b_ref, o_ref):
    o_ref[...] = p_ref[0] + p_ref[1] + b_ref[...]


def _combine(partials, bias2d):
    return pl.pallas_call(
        _comb_body,
        grid=(10,),
        in_specs=[
            pl.BlockSpec((NC, 1000, D), lambda i: (0, i, 0)),
            pl.BlockSpec((1, D), lambda i: (0, 0)),
        ],
        out_specs=pl.BlockSpec((1000, D), lambda i: (i, 0)),
        out_shape=jax.ShapeDtypeStruct((N, D), jnp.float32),
    )(partials, bias2d)


def kernel(x, edge_index_0, edge_index_1, edge_index_2,
           attn_0, attn_1, attn_2, net_weights, W, bias):
    xw = _matmul(x, W)
    nw_flat = jnp.pad(net_weights, ((0, 0), (0, 8 - NMOD))).reshape(64)
    partials = _sc_scatter(xw, edge_index_0, edge_index_1, edge_index_2,
                           attn_0, attn_1, attn_2, nw_flat)
    return _combine(partials, bias.reshape(1, D))


# speedup vs baseline: 34.2261x; 34.2261x over previous
import functools

import jax
import jax.numpy as jnp
from jax import lax
from jax.experimental import pallas as pl
from jax.experimental.pallas import tpu as pltpu
from jax.experimental.pallas import tpu_sc as plsc

N = 10000
E = 160000
D = 128
HEADS = 8
DIM = 16
NMOD = 3

NC = 2
NS = 16
CHUNK = 40
CPAD = 48
NPAD = N + 64
E_PER_WORKER = E // (NC * NS)
NCHUNK = E_PER_WORKER // CHUNK
ROWS_PER_TILE = 624
RB = 48
TAIL = N - NS * ROWS_PER_TILE


def _mm_body(x_ref, w_ref, o_ref):
    o_ref[...] = jnp.dot(x_ref[...], w_ref[...],
                         preferred_element_type=jnp.float32)


def _matmul(x, w):
    return pl.pallas_call(
        _mm_body,
        grid=(10,),
        in_specs=[
            pl.BlockSpec((1000, D), lambda j: (j, 0)),
            pl.BlockSpec((D, D), lambda j: (0, 0)),
        ],
        out_specs=pl.BlockSpec((1000, D), lambda j: (j, 0)),
        out_shape=jax.ShapeDtypeStruct((N, D), jnp.float32),
    )(x, w)


def _expand(attn, exp_m):
    return pl.pallas_call(
        _mm_body,
        grid=(80,),
        in_specs=[
            pl.BlockSpec((2000, HEADS), lambda j: (j, 0)),
            pl.BlockSpec((HEADS, D), lambda j: (0, 0)),
        ],
        out_specs=pl.BlockSpec((2000, D), lambda j: (j, 0)),
        out_shape=jax.ShapeDtypeStruct((E, D), jnp.float32),
    )(attn, exp_m)


def _sc_body(xw_hbm, s0, d0, s1, d1, s2, d2, w0, w1, w2, out_hbm,
             src_v, dst_v, wexp_v, rows_v, rows2_v, bounce_v, acc, sem):
    c = lax.axis_index("c")
    s = lax.axis_index("s")
    iota = lax.iota(jnp.int32, 16)

    src_v[pl.ds(32, 16)] = jnp.where(iota < 8, 0, iota - 8)
    dst_v[pl.ds(32, 16)] = N + 32 + iota

    zeros16 = jnp.zeros((16,), jnp.float32)
    for r in range(CHUNK, CPAD):
        for j in range(D // 16):
            rows2_v[r, pl.ds(16 * j, 16)] = zeros16

    def _zrow(i, _):
        for j in range(D // 16):
            bounce_v[i, pl.ds(16 * j, 16)] = zeros16
        return 0

    lax.fori_loop(0, RB, _zrow, 0)
    row_base = s * ROWS_PER_TILE
    for k in range(ROWS_PER_TILE // RB):
        pltpu.sync_copy(bounce_v, acc.at[pl.ds(row_base + RB * k, RB)])

    @pl.when(s == NS - 1)
    def _zero_tail():
        pltpu.sync_copy(bounce_v.at[pl.ds(0, TAIL)],
                        acc.at[pl.ds(NS * ROWS_PER_TILE, TAIL)])

    plsc.subcore_barrier()

    srcs = [s0, s1, s2]
    dsts = [d0, d1, d2]
    wexps = [w0, w1, w2]
    for m in range(NMOD):
        s_hbm = srcs[m]
        d_hbm = dsts[m]
        w_hbm = wexps[m]
        base_e = c * (E // NC) + s * E_PER_WORKER

        def _chunk(k, _, s_hbm=s_hbm, d_hbm=d_hbm, w_hbm=w_hbm,
                   base_e=base_e):
            start = base_e + k * CHUNK
            pltpu.sync_copy(s_hbm.at[pl.ds(start, CHUNK)],
                            src_v.at[pl.ds(0, CHUNK)])
            pltpu.sync_copy(d_hbm.at[pl.ds(start, CHUNK)],
                            dst_v.at[pl.ds(0, CHUNK)])
            pltpu.sync_copy(w_hbm.at[pl.ds(start, CHUNK), :], wexp_v)
            pltpu.async_copy(xw_hbm.at[src_v], rows_v, sem).wait()

            def _edge(i, _):
                for j in range(D // 16):
                    sl = pl.ds(16 * j, 16)
                    rows2_v[i, sl] = rows_v[i, sl] * wexp_v[i, sl]
                return 0

            lax.fori_loop(0, CHUNK, _edge, 0)
            pltpu.sync_copy(rows2_v, acc.at[dst_v], add=True)
            return 0

        lax.fori_loop(0, NCHUNK, _chunk, 0)

    plsc.subcore_barrier()
    for k in range(ROWS_PER_TILE // RB):
        pltpu.sync_copy(acc.at[pl.ds(row_base + RB * k, RB)], bounce_v)
        pltpu.sync_copy(bounce_v,
                        out_hbm.at[c, pl.ds(row_base + RB * k, RB)])

    @pl.when(s == NS - 1)
    def _read_tail():
        pltpu.sync_copy(acc.at[pl.ds(NS * ROWS_PER_TILE, TAIL)],
                        bounce_v.at[pl.ds(0, TAIL)])
        pltpu.sync_copy(bounce_v.at[pl.ds(0, TAIL)],
                        out_hbm.at[c, pl.ds(NS * ROWS_PER_TILE, TAIL)])


@functools.lru_cache(maxsize=1)
def _make_sc_scatter():
    return pl.kernel(
        _sc_body,
        out_type=jax.ShapeDtypeStruct((NC, N, D), jnp.float32),
        mesh=plsc.VectorSubcoreMesh(core_axis_name="c", subcore_axis_name="s",
                                    num_cores=NC, num_subcores=NS),
        compiler_params=pltpu.CompilerParams(needs_layout_passes=False),
        scratch_types=[
            pltpu.VMEM((CPAD,), jnp.int32),
            pltpu.VMEM((CPAD,), jnp.int32),
            pltpu.VMEM((CHUNK, D), jnp.float32),
            pltpu.VMEM((CPAD, D), jnp.float32),
            pltpu.VMEM((CPAD, D), jnp.float32),
            pltpu.VMEM((RB, D), jnp.float32),
            pltpu.VMEM_SHARED((NPAD, D), jnp.float32),
            pltpu.SemaphoreType.DMA,
        ],
    )


def _comb_body(p_ref, b_ref, o_ref):
    o_ref[...] = p_ref[0] + p_ref[1] + b_ref[...]


def _combine(partials, bias2d):
    return pl.pallas_call(
        _comb_body,
        grid=(10,),
        in_specs=[
            pl.BlockSpec((NC, 1000, D), lambda i: (0, i, 0)),
            pl.BlockSpec((1, D), lambda i: (0, 0)),
        ],
        out_specs=pl.BlockSpec((1000, D), lambda i: (i, 0)),
        out_shape=jax.ShapeDtypeStruct((N, D), jnp.float32),
    )(partials, bias2d)


def kernel(x, edge_index_0, edge_index_1, edge_index_2,
           attn_0, attn_1, attn_2, net_weights, W, bias):
    xw = _matmul(x, W)
    col_head = jnp.arange(D, dtype=jnp.int32) // DIM
    head = jnp.arange(HEADS, dtype=jnp.int32)
    blockmask = (head[:, None] == col_head[None, :]).astype(jnp.float32)
    attns = [attn_0, attn_1, attn_2]
    wexps = [
        _expand(attns[m], blockmask * net_weights[:, m][:, None])
        for m in range(NMOD)
    ]
    partials = _make_sc_scatter()(
        xw,
        edge_index_0[0], edge_index_0[1],
        edge_index_1[0], edge_index_1[1],
        edge_index_2[0], edge_index_2[1],
        wexps[0], wexps[1], wexps[2])
    return _combine(partials, bias.reshape(1, D))


# speedup vs baseline: 47.5920x; 1.3905x over previous
import functools

import jax
import jax.numpy as jnp
from jax import lax
from jax.experimental import pallas as pl
from jax.experimental.pallas import tpu as pltpu
from jax.experimental.pallas import tpu_sc as plsc

N = 10000
E = 160000
D = 128
HEADS = 8
DIM = 16
NMOD = 3

NC = 2
NS = 16
CHUNK = 40
BATCH = 5
BE = CHUNK * BATCH
E_PER_WORKER = E // (NC * NS)
NBATCH = E_PER_WORKER // BE
ROWS_PER_TILE = 624
RB = 48
TAIL = N - NS * ROWS_PER_TILE


def _mm_body(x_ref, w_ref, o_ref):
    o_ref[...] = jnp.dot(x_ref[...], w_ref[...],
                         preferred_element_type=jnp.float32)


def _matmul(x, w):
    return pl.pallas_call(
        _mm_body,
        grid=(10,),
        in_specs=[
            pl.BlockSpec((1000, D), lambda j: (j, 0)),
            pl.BlockSpec((D, D), lambda j: (0, 0)),
        ],
        out_specs=pl.BlockSpec((1000, D), lambda j: (j, 0)),
        out_shape=jax.ShapeDtypeStruct((N, D), jnp.float32),
    )(x, w)


def _expand(attn, exp_m):
    return pl.pallas_call(
        _mm_body,
        grid=(80,),
        in_specs=[
            pl.BlockSpec((2000, HEADS), lambda j: (j, 0)),
            pl.BlockSpec((HEADS, D), lambda j: (0, 0)),
        ],
        out_specs=pl.BlockSpec((2000, D), lambda j: (j, 0)),
        out_shape=jax.ShapeDtypeStruct((E, D), jnp.float32),
    )(attn, exp_m)


def _sc_body(xw_hbm, s0, d0, s1, d1, s2, d2, w0, w1, w2, out_hbm,
             src_v, dst_v, wexp_v, rows_v, rows2_v, bounce_v, acc, sem):
    c = lax.axis_index("c")
    s = lax.axis_index("s")

    zeros16 = jnp.zeros((16,), jnp.float32)

    def _zrow(i, _):
        for j in range(D // 16):
            bounce_v[i, pl.ds(16 * j, 16)] = zeros16
        return 0

    lax.fori_loop(0, RB, _zrow, 0)
    row_base = s * ROWS_PER_TILE
    for k in range(ROWS_PER_TILE // RB):
        pltpu.sync_copy(bounce_v, acc.at[pl.ds(row_base + RB * k, RB)])

    @pl.when(s == NS - 1)
    def _zero_tail():
        pltpu.sync_copy(bounce_v.at[pl.ds(0, TAIL)],
                        acc.at[pl.ds(NS * ROWS_PER_TILE, TAIL)])

    plsc.subcore_barrier()

    srcs = [s0, s1, s2]
    dsts = [d0, d1, d2]
    wexps = [w0, w1, w2]
    for m in range(NMOD):
        s_hbm = srcs[m]
        d_hbm = dsts[m]
        w_hbm = wexps[m]
        base_e = c * (E // NC) + s * E_PER_WORKER

        def _batch(kb, _, s_hbm=s_hbm, d_hbm=d_hbm, w_hbm=w_hbm,
                   base_e=base_e):
            start = base_e + kb * BE
            pltpu.sync_copy(s_hbm.at[pl.ds(start, BE)], src_v)
            pltpu.sync_copy(d_hbm.at[pl.ds(start, BE)], dst_v)
            pltpu.sync_copy(w_hbm.at[pl.ds(start, BE), :], wexp_v)
            for t in range(BATCH):
                pltpu.async_copy(
                    xw_hbm.at[src_v.at[pl.ds(t * CHUNK, CHUNK)]],
                    rows_v, sem).wait()

                def _edge(i, _, t=t):
                    for j in range(D // 16):
                        sl = pl.ds(16 * j, 16)
                        rows2_v[i, sl] = (rows_v[i, sl]
                                          * wexp_v[t * CHUNK + i, sl])
                    return 0

                lax.fori_loop(0, CHUNK, _edge, 0)
                pltpu.sync_copy(rows2_v,
                                acc.at[dst_v.at[pl.ds(t * CHUNK, CHUNK)]],
                                add=True)
            return 0

        lax.fori_loop(0, NBATCH, _batch, 0)

    plsc.subcore_barrier()
    for k in range(ROWS_PER_TILE // RB):
        pltpu.sync_copy(acc.at[pl.ds(row_base + RB * k, RB)], bounce_v)
        pltpu.sync_copy(bounce_v,
                        out_hbm.at[c, pl.ds(row_base + RB * k, RB)])

    @pl.when(s == NS - 1)
    def _read_tail():
        pltpu.sync_copy(acc.at[pl.ds(NS * ROWS_PER_TILE, TAIL)],
                        bounce_v.at[pl.ds(0, TAIL)])
        pltpu.sync_copy(bounce_v.at[pl.ds(0, TAIL)],
                        out_hbm.at[c, pl.ds(NS * ROWS_PER_TILE, TAIL)])


@functools.lru_cache(maxsize=1)
def _make_sc_scatter():
    return pl.kernel(
        _sc_body,
        out_type=jax.ShapeDtypeStruct((NC, N, D), jnp.float32),
        mesh=plsc.VectorSubcoreMesh(core_axis_name="c", subcore_axis_name="s",
                                    num_cores=NC, num_subcores=NS),
        compiler_params=pltpu.CompilerParams(needs_layout_passes=False),
        scratch_types=[
            pltpu.VMEM((BE,), jnp.int32),
            pltpu.VMEM((BE,), jnp.int32),
            pltpu.VMEM((BE, D), jnp.float32),
            pltpu.VMEM((CHUNK, D), jnp.float32),
            pltpu.VMEM((CHUNK, D), jnp.float32),
            pltpu.VMEM((RB, D), jnp.float32),
            pltpu.VMEM_SHARED((N, D), jnp.float32),
            pltpu.SemaphoreType.DMA,
        ],
    )


def _comb_body(p_ref, b_ref, o_ref):
    o_ref[...] = p_ref[0] + p_ref[1] + b_ref[...]


def _combine(partials, bias2d):
    return pl.pallas_call(
        _comb_body,
        grid=(10,),
        in_specs=[
            pl.BlockSpec((NC, 1000, D), lambda i: (0, i, 0)),
            pl.BlockSpec((1, D), lambda i: (0, 0)),
        ],
        out_specs=pl.BlockSpec((1000, D), lambda i: (i, 0)),
        out_shape=jax.ShapeDtypeStruct((N, D), jnp.float32),
    )(partials, bias2d)


def kernel(x, edge_index_0, edge_index_1, edge_index_2,
           attn_0, attn_1, attn_2, net_weights, W, bias):
    xw = _matmul(x, W)
    col_head = jnp.arange(D, dtype=jnp.int32) // DIM
    head = jnp.arange(HEADS, dtype=jnp.int32)
    blockmask = (head[:, None] == col_head[None, :]).astype(jnp.float32)
    attns = [attn_0, attn_1, attn_2]
    wexps = [
        _expand(attns[m], blockmask * net_weights[:, m][:, None])
        for m in range(NMOD)
    ]
    partials = _make_sc_scatter()(
        xw,
        edge_index_0[0], edge_index_0[1],
        edge_index_1[0], edge_index_1[1],
        edge_index_2[0], edge_index_2[1],
        wexps[0], wexps[1], wexps[2])
    return _combine(partials, bias.reshape(1, D))


# speedup vs baseline: 56.2358x; 1.1816x over previous
import functools

import jax
import jax.numpy as jnp
from jax import lax
from jax.experimental import pallas as pl
from jax.experimental.pallas import tpu as pltpu
from jax.experimental.pallas import tpu_sc as plsc

N = 10000
E = 160000
D = 128
HEADS = 8
DIM = 16
NMOD = 3

NC = 2
NS = 16
CHUNK = 40
BATCH = 5
BE = CHUNK * BATCH
E_PER_WORKER = E // (NC * NS)
NBATCH = E_PER_WORKER // BE
ROWS_PER_TILE = 624
RB = 48
TAIL = N - NS * ROWS_PER_TILE


def _mm_body(x_ref, w_ref, o_ref):
    o_ref[...] = jnp.dot(x_ref[...], w_ref[...],
                         preferred_element_type=jnp.float32)


def _matmul(x, w):
    return pl.pallas_call(
        _mm_body,
        grid=(10,),
        in_specs=[
            pl.BlockSpec((1000, D), lambda j: (j, 0)),
            pl.BlockSpec((D, D), lambda j: (0, 0)),
        ],
        out_specs=pl.BlockSpec((1000, D), lambda j: (j, 0)),
        out_shape=jax.ShapeDtypeStruct((N, D), jnp.float32),
    )(x, w)


def _expand(attn, exp_m):
    return pl.pallas_call(
        _mm_body,
        grid=(80,),
        in_specs=[
            pl.BlockSpec((2000, HEADS), lambda j: (j, 0)),
            pl.BlockSpec((HEADS, D), lambda j: (0, 0)),
        ],
        out_specs=pl.BlockSpec((2000, D), lambda j: (j, 0)),
        out_shape=jax.ShapeDtypeStruct((E, D), jnp.float32),
    )(attn, exp_m)


def _sc_body(xw_hbm, s0, d0, s1, d1, s2, d2, w0, w1, w2, out_hbm,
             src_v, dst_v, wexp_v, rows_v, rows_vb, rows2_v, bounce_v, acc,
             sem):
    c = lax.axis_index("c")
    s = lax.axis_index("s")

    zeros16 = jnp.zeros((16,), jnp.float32)

    def _zrow(i, _):
        for j in range(D // 16):
            bounce_v[i, pl.ds(16 * j, 16)] = zeros16
        return 0

    lax.fori_loop(0, RB, _zrow, 0)
    row_base = s * ROWS_PER_TILE
    for k in range(ROWS_PER_TILE // RB):
        pltpu.sync_copy(bounce_v, acc.at[pl.ds(row_base + RB * k, RB)])

    @pl.when(s == NS - 1)
    def _zero_tail():
        pltpu.sync_copy(bounce_v.at[pl.ds(0, TAIL)],
                        acc.at[pl.ds(NS * ROWS_PER_TILE, TAIL)])

    plsc.subcore_barrier()

    srcs = [s0, s1, s2]
    dsts = [d0, d1, d2]
    wexps = [w0, w1, w2]
    for m in range(NMOD):
        s_hbm = srcs[m]
        d_hbm = dsts[m]
        w_hbm = wexps[m]
        base_e = c * (E // NC) + s * E_PER_WORKER

        def _batch(kb, _, s_hbm=s_hbm, d_hbm=d_hbm, w_hbm=w_hbm,
                   base_e=base_e):
            start = base_e + kb * BE
            pltpu.sync_copy(s_hbm.at[pl.ds(start, BE)], src_v)
            pltpu.sync_copy(d_hbm.at[pl.ds(start, BE)], dst_v)
            pltpu.sync_copy(w_hbm.at[pl.ds(start, BE), :], wexp_v)
            rbufs = (rows_v, rows_vb)
            h = pltpu.async_copy(
                xw_hbm.at[src_v.at[pl.ds(0, CHUNK)]], rbufs[0], sem)
            for t in range(BATCH):
                h.wait()
                if t + 1 < BATCH:
                    h = pltpu.async_copy(
                        xw_hbm.at[src_v.at[pl.ds((t + 1) * CHUNK, CHUNK)]],
                        rbufs[(t + 1) % 2], sem)

                def _edge(i, _, t=t):
                    for j in range(D // 16):
                        sl = pl.ds(16 * j, 16)
                        rows2_v[i, sl] = (rbufs[t % 2][i, sl]
                                          * wexp_v[t * CHUNK + i, sl])
                    return 0

                lax.fori_loop(0, CHUNK, _edge, 0)
                pltpu.sync_copy(rows2_v,
                                acc.at[dst_v.at[pl.ds(t * CHUNK, CHUNK)]],
                                add=True)
            return 0

        lax.fori_loop(0, NBATCH, _batch, 0)

    plsc.subcore_barrier()
    for k in range(ROWS_PER_TILE // RB):
        pltpu.sync_copy(acc.at[pl.ds(row_base + RB * k, RB)], bounce_v)
        pltpu.sync_copy(bounce_v,
                        out_hbm.at[c, pl.ds(row_base + RB * k, RB)])

    @pl.when(s == NS - 1)
    def _read_tail():
        pltpu.sync_copy(acc.at[pl.ds(NS * ROWS_PER_TILE, TAIL)],
                        bounce_v.at[pl.ds(0, TAIL)])
        pltpu.sync_copy(bounce_v.at[pl.ds(0, TAIL)],
                        out_hbm.at[c, pl.ds(NS * ROWS_PER_TILE, TAIL)])


@functools.lru_cache(maxsize=1)
def _make_sc_scatter():
    return pl.kernel(
        _sc_body,
        out_type=jax.ShapeDtypeStruct((NC, N, D), jnp.float32),
        mesh=plsc.VectorSubcoreMesh(core_axis_name="c", subcore_axis_name="s",
                                    num_cores=NC, num_subcores=NS),
        compiler_params=pltpu.CompilerParams(needs_layout_passes=False),
        scratch_types=[
            pltpu.VMEM((BE,), jnp.int32),
            pltpu.VMEM((BE,), jnp.int32),
            pltpu.VMEM((BE, D), jnp.float32),
            pltpu.VMEM((CHUNK, D), jnp.float32),
            pltpu.VMEM((CHUNK, D), jnp.float32),
            pltpu.VMEM((CHUNK, D), jnp.float32),
            pltpu.VMEM((RB, D), jnp.float32),
            pltpu.VMEM_SHARED((N, D), jnp.float32),
            pltpu.SemaphoreType.DMA,
        ],
    )


def _comb_body(p_ref, b_ref, o_ref):
    o_ref[...] = p_ref[0] + p_ref[1] + b_ref[...]


def _combine(partials, bias2d):
    return pl.pallas_call(
        _comb_body,
        grid=(10,),
        in_specs=[
            pl.BlockSpec((NC, 1000, D), lambda i: (0, i, 0)),
            pl.BlockSpec((1, D), lambda i: (0, 0)),
        ],
        out_specs=pl.BlockSpec((1000, D), lambda i: (i, 0)),
        out_shape=jax.ShapeDtypeStruct((N, D), jnp.float32),
    )(partials, bias2d)


def kernel(x, edge_index_0, edge_index_1, edge_index_2,
           attn_0, attn_1, attn_2, net_weights, W, bias):
    xw = _matmul(x, W)
    col_head = jnp.arange(D, dtype=jnp.int32) // DIM
    head = jnp.arange(HEADS, dtype=jnp.int32)
    blockmask = (head[:, None] == col_head[None, :]).astype(jnp.float32)
    attns = [attn_0, attn_1, attn_2]
    wexps = [
        _expand(attns[m], blockmask * net_weights[:, m][:, None])
        for m in range(NMOD)
    ]
    partials = _make_sc_scatter()(
        xw,
        edge_index_0[0], edge_index_0[1],
        edge_index_1[0], edge_index_1[1],
        edge_index_2[0], edge_index_2[1],
        wexps[0], wexps[1], wexps[2])
    return _combine(partials, bias.reshape(1, D))
